# Initial kernel scaffold; baseline (speedup 1.0000x reference)
import jax
import jax.numpy as jnp
from jax.experimental import pallas as pl


def kernel(x, nodes, nbr1, nbr2, W0l, b0l, W0r, b0r, W1l, b1l, W1r, b1r, Wp, bp):
    raise NotImplementedError("write your pallas kernel here")


# speedup vs baseline: 3.9334x; 3.9334x over previous
import functools

import jax
import jax.numpy as jnp
from jax import lax
from jax.experimental import pallas as pl
from jax.experimental.pallas import tpu as pltpu
from jax.experimental.pallas import tpu_sc as plsc

N_NODES = 100000
D = 128
B = 4096
T = 5
S1, S2 = 5, 2
HID0, HID1 = 128, 64
NCLS = 16

NC, NS = 2, 16
NW = NC * NS
CH = 128

N1 = T * S1 * B
N2 = T * S2 * S1 * B
CH0 = (B // NW) // CH
CH1 = (N1 // NW) // CH
CH2 = (N2 // NW) // CH


def _sc_gather_body(x_hbm, idx0_hbm, idx1_hbm, idx2_hbm,
                    out0_hbm, out1_hbm, out2_hbm,
                    idx_v, rows_v, sem):
    wid = lax.axis_index("s") * NC + lax.axis_index("c")

    def run(idx_hbm, out_hbm, nchunks):
        base_w = wid * (nchunks * CH)

        def body(k, carry):
            base = base_w + k * CH
            pltpu.sync_copy(idx_hbm.at[pl.ds(base, CH)], idx_v)
            pltpu.async_copy(x_hbm.at[idx_v], rows_v, sem).wait()
            pltpu.sync_copy(rows_v, out_hbm.at[pl.ds(base, CH)])
            return carry

        lax.fori_loop(0, nchunks, body, 0)

    run(idx0_hbm, out0_hbm, CH0)
    run(idx1_hbm, out1_hbm, CH1)
    run(idx2_hbm, out2_hbm, CH2)


def _sc_gather(x, idx0, idx1, idx2):
    gather = functools.partial(
        pl.kernel,
        out_type=(
            jax.ShapeDtypeStruct((B, D), jnp.float32),
            jax.ShapeDtypeStruct((N1, D), jnp.float32),
            jax.ShapeDtypeStruct((N2, D), jnp.float32),
        ),
        mesh=plsc.VectorSubcoreMesh(core_axis_name="c", subcore_axis_name="s",
                                    num_cores=NC, num_subcores=NS),
        scratch_types=[
            pltpu.VMEM((CH,), jnp.int32),
            pltpu.VMEM((CH, D), jnp.float32),
            pltpu.SemaphoreType.DMA,
        ],
    )(_sc_gather_body)
    return gather(x, idx0, idx1, idx2)


def _tc_body(h0_ref, h1_ref, h2_ref, w0l_ref, w0r_ref, w1l_ref, w1r_ref,
             wp_ref, b0_ref, b1_ref, bp_ref, out_ref):
    t = pl.program_id(1)
    f32 = jnp.float32

    w0l = w0l_ref[...]
    w0r = w0r_ref[...]
    b0 = b0_ref[...]

    h1s = [h1_ref[0, j] for j in range(S1)]
    h1m = (h1s[0] + h1s[1] + h1s[2] + h1s[3] + h1s[4]) / 5.0

    pre = (jnp.dot(h0_ref[...], w0l, preferred_element_type=f32)
           + jnp.dot(h1m, w0r, preferred_element_type=f32) + b0)
    s0_seed = (pre > 1.0).astype(f32)

    acc = None
    for j in range(S1):
        h2m = (h2_ref[0, j] + h2_ref[0, S1 + j]) / 2.0
        pre_j = (jnp.dot(h1s[j], w0l, preferred_element_type=f32)
                 + jnp.dot(h2m, w0r, preferred_element_type=f32) + b0)
        sj = (pre_j > 1.0).astype(f32)
        acc = sj if acc is None else acc + sj
    s0n_mean = acc / 5.0

    pre1 = (jnp.dot(s0_seed, w1l_ref[...], preferred_element_type=f32)
            + jnp.dot(s0n_mean, w1r_ref[...], preferred_element_type=f32)
            + b1_ref[...])
    s1 = (pre1 > 1.0).astype(f32)

    contrib = jnp.dot(s1, wp_ref[0], preferred_element_type=f32)

    @pl.when(t == 0)
    def _init():
        out_ref[...] = bp_ref[...] + contrib

    @pl.when(t != 0)
    def _acc():
        out_ref[...] += contrib


def _tc_net(h0, h1g, h2r, W0l, W0r, W1l, W1r, wpt, b0, b1, bp2, block_b):
    nb = B // block_b
    grid = (nb, T)
    return pl.pallas_call(
        _tc_body,
        grid=grid,
        in_specs=[
            pl.BlockSpec((block_b, D), lambda i, t: (i, 0)),
            pl.BlockSpec((1, S1, block_b, D), lambda i, t: (t, 0, i, 0)),
            pl.BlockSpec((1, S2 * S1, block_b, D), lambda i, t: (t, 0, i, 0)),
            pl.BlockSpec((D, HID0), lambda i, t: (0, 0)),
            pl.BlockSpec((D, HID0), lambda i, t: (0, 0)),
            pl.BlockSpec((HID0, HID1), lambda i, t: (0, 0)),
            pl.BlockSpec((HID0, HID1), lambda i, t: (0, 0)),
            pl.BlockSpec((1, HID1, NCLS), lambda i, t: (t, 0, 0)),
            pl.BlockSpec((1, HID0), lambda i, t: (0, 0)),
            pl.BlockSpec((1, HID1), lambda i, t: (0, 0)),
            pl.BlockSpec((1, NCLS), lambda i, t: (0, 0)),
        ],
        out_specs=pl.BlockSpec((block_b, NCLS), lambda i, t: (i, 0)),
        out_shape=jax.ShapeDtypeStruct((B, NCLS), jnp.float32),
    )(h0, h1g, h2r, W0l, W0r, W1l, W1r, wpt, b0, b1, bp2)


def kernel(x, nodes, nbr1, nbr2, W0l, b0l, W0r, b0r, W1l, b1l, W1r, b1r, Wp, bp):
    idx1 = nbr1.reshape(T, B, S1).transpose(0, 2, 1).reshape(-1)
    idx2 = nbr2.reshape(T, B, S1, S2).transpose(0, 3, 2, 1).reshape(-1)

    h0, h1f, h2f = _sc_gather(x, nodes, idx1, idx2)
    h1g = h1f.reshape(T, S1, B, D)
    h2r = h2f.reshape(T, S2 * S1, B, D)

    b0 = (b0l + b0r).reshape(1, HID0)
    b1 = (b1l + b1r).reshape(1, HID1)
    bp2 = bp.reshape(1, NCLS)
    wpt = Wp.reshape(T, HID1, NCLS)

    return _tc_net(h0, h1g, h2r, W0l, W0r, W1l, W1r, wpt, b0, b1, bp2,
                   block_b=1024)


# speedup vs baseline: 5.3780x; 1.3673x over previous
import functools

import jax
import jax.numpy as jnp
from jax import lax
from jax.experimental import pallas as pl
from jax.experimental.pallas import tpu as pltpu
from jax.experimental.pallas import tpu_sc as plsc

N_NODES = 100000
D = 128
B = 4096
T = 5
S1, S2 = 5, 2
HID0, HID1 = 128, 64
NCLS = 16

NC, NS = 2, 16
NW = NC * NS
CH = 128

N1 = T * S1 * B
N2 = T * S2 * S1 * B
NG = B + N1 + N2
PER_W = NG // NW
NCHUNK = PER_W // CH
OFF1 = B
OFF2 = B + N1


def _sc_gather_body(x_hbm, idx_hbm, out_hbm, iv0, iv1, rb0, rb1,
                    si0, si1, sg0, sg1, sw0, sw1):
    wid = lax.axis_index("s") * NC + lax.axis_index("c")
    base_w = wid * PER_W

    iv = (iv0, iv1)
    rb = (rb0, rb1)
    si = (si0, si1)
    sg = (sg0, sg1)
    sw = (sw0, sw1)

    def load_idx(i, b):
        pltpu.async_copy(idx_hbm.at[pl.ds(base_w + i * CH, CH)], iv[b], si[b])

    def gather(i, b):
        del i
        pltpu.async_copy(x_hbm.at[iv[b]], rb[b], sg[b])

    def writeback(i, b):
        pltpu.async_copy(rb[b], out_hbm.at[pl.ds(base_w + i * CH, CH)], sw[b])

    def wait_idx(b):
        pltpu.make_async_copy(idx_hbm.at[pl.ds(0, CH)], iv[b], si[b]).wait()

    def wait_gather(b):
        pltpu.make_async_copy(x_hbm.at[iv[b]], rb[b], sg[b]).wait()

    def wait_write(b):
        pltpu.make_async_copy(rb[b], out_hbm.at[pl.ds(0, CH)], sw[b]).wait()

    load_idx(0, 0)
    load_idx(1, 1)
    wait_idx(0)
    gather(0, 0)
    wait_idx(1)
    gather(1, 1)
    wait_gather(0)
    writeback(0, 0)
    load_idx(2, 0)

    def pair(g, carry):
        i0 = 2 * g
        i1 = i0 + 1
        wait_write(0)
        wait_idx(0)
        gather(i0, 0)
        wait_gather(1)
        writeback(i0 - 1, 1)
        load_idx(i1, 1)
        wait_write(1)
        wait_idx(1)
        gather(i1, 1)
        wait_gather(0)
        writeback(i0, 0)
        load_idx(i1 + 1, 0)
        return carry

    lax.fori_loop(1, NCHUNK // 2, pair, 0)

    wait_gather(1)
    writeback(NCHUNK - 1, 1)
    wait_idx(0)
    wait_write(0)
    wait_write(1)


def _sc_gather(x, idx_all):
    gather = functools.partial(
        pl.kernel,
        out_type=jax.ShapeDtypeStruct((NG, D), jnp.float32),
        mesh=plsc.VectorSubcoreMesh(core_axis_name="c", subcore_axis_name="s",
                                    num_cores=NC, num_subcores=NS),
        scratch_types=[
            pltpu.VMEM((CH,), jnp.int32),
            pltpu.VMEM((CH,), jnp.int32),
            pltpu.VMEM((CH, D), jnp.float32),
            pltpu.VMEM((CH, D), jnp.float32),
            pltpu.SemaphoreType.DMA,
            pltpu.SemaphoreType.DMA,
            pltpu.SemaphoreType.DMA,
            pltpu.SemaphoreType.DMA,
            pltpu.SemaphoreType.DMA,
            pltpu.SemaphoreType.DMA,
        ],
    )(_sc_gather_body)
    return gather(x, idx_all)


def _tc_body(*refs):
    (h0_ref, h1r0, h1r1, h1r2, h1r3, h1r4,
     h2r0, h2r1, h2r2, h2r3, h2r4, h2r5, h2r6, h2r7, h2r8, h2r9,
     w0l_ref, w0r_ref, w1l_ref, w1r_ref, wp_ref, b0_ref, b1_ref, bp_ref,
     out_ref) = refs
    t = pl.program_id(1)
    f32 = jnp.float32
    h1_refs = (h1r0, h1r1, h1r2, h1r3, h1r4)
    h2_refs = (h2r0, h2r1, h2r2, h2r3, h2r4, h2r5, h2r6, h2r7, h2r8, h2r9)

    w0l = w0l_ref[...]
    w0r = w0r_ref[...]
    b0 = b0_ref[...]

    h1s = [r[...] for r in h1_refs]
    h1m = (h1s[0] + h1s[1] + h1s[2] + h1s[3] + h1s[4]) / 5.0

    pre = (jnp.dot(h0_ref[...], w0l, preferred_element_type=f32)
           + jnp.dot(h1m, w0r, preferred_element_type=f32) + b0)
    s0_seed = (pre > 1.0).astype(f32)

    acc = None
    for j in range(S1):
        h2m = (h2_refs[j][...] + h2_refs[S1 + j][...]) / 2.0
        pre_j = (jnp.dot(h1s[j], w0l, preferred_element_type=f32)
                 + jnp.dot(h2m, w0r, preferred_element_type=f32) + b0)
        sj = (pre_j > 1.0).astype(f32)
        acc = sj if acc is None else acc + sj
    s0n_mean = acc / 5.0

    pre1 = (jnp.dot(s0_seed, w1l_ref[...], preferred_element_type=f32)
            + jnp.dot(s0n_mean, w1r_ref[...], preferred_element_type=f32)
            + b1_ref[...])
    s1 = (pre1 > 1.0).astype(f32)

    contrib = jnp.dot(s1, wp_ref[0], preferred_element_type=f32)

    @pl.when(t == 0)
    def _init():
        out_ref[...] = bp_ref[...] + contrib

    @pl.when(t != 0)
    def _acc():
        out_ref[...] += contrib


def _tc_net(g, W0l, W0r, W1l, W1r, wpt, b0, b1, bp2, block_b):
    nb = B // block_b
    grid = (nb, T)
    blk = B // block_b

    def h1_map(j):
        return lambda i, t, j=j: (blk + (t * S1 + j) * blk + i, 0)

    def h2_map(q):
        return lambda i, t, q=q: (OFF2 // block_b + (t * S1 * S2 + q) * blk + i, 0)

    slab = pl.BlockSpec((block_b, D), lambda i, t: (i, 0))
    in_specs = (
        [slab]
        + [pl.BlockSpec((block_b, D), h1_map(j)) for j in range(S1)]
        + [pl.BlockSpec((block_b, D), h2_map(q)) for q in range(S1 * S2)]
        + [
            pl.BlockSpec((D, HID0), lambda i, t: (0, 0)),
            pl.BlockSpec((D, HID0), lambda i, t: (0, 0)),
            pl.BlockSpec((HID0, HID1), lambda i, t: (0, 0)),
            pl.BlockSpec((HID0, HID1), lambda i, t: (0, 0)),
            pl.BlockSpec((1, HID1, NCLS), lambda i, t: (t, 0, 0)),
            pl.BlockSpec((1, HID0), lambda i, t: (0, 0)),
            pl.BlockSpec((1, HID1), lambda i, t: (0, 0)),
            pl.BlockSpec((1, NCLS), lambda i, t: (0, 0)),
        ]
    )
    args = ([g] * 16) + [W0l, W0r, W1l, W1r, wpt, b0, b1, bp2]
    return pl.pallas_call(
        _tc_body,
        grid=grid,
        in_specs=in_specs,
        out_specs=pl.BlockSpec((block_b, NCLS), lambda i, t: (i, 0)),
        out_shape=jax.ShapeDtypeStruct((B, NCLS), jnp.float32),
    )(*args)


def kernel(x, nodes, nbr1, nbr2, W0l, b0l, W0r, b0r, W1l, b1l, W1r, b1r, Wp, bp):
    idx1 = nbr1.reshape(T, B, S1).transpose(0, 2, 1).reshape(-1)
    idx2 = nbr2.reshape(T, B, S1, S2).transpose(0, 3, 2, 1).reshape(-1)
    idx_all = jnp.concatenate(
        [nodes, idx1, idx2, jnp.zeros((CH,), jnp.int32)])

    g = _sc_gather(x, idx_all)

    b0 = (b0l + b0r).reshape(1, HID0)
    b1 = (b1l + b1r).reshape(1, HID1)
    bp2 = bp.reshape(1, NCLS)
    wpt = Wp.reshape(T, HID1, NCLS)

    return _tc_net(g, W0l, W0r, W1l, W1r, wpt, b0, b1, bp2, block_b=1024)
